# baseline (speedup 1.0000x reference)
import functools

import jax
import jax.numpy as jnp
from jax import lax
from jax.experimental import pallas as pl
from jax.experimental.pallas import tpu as pltpu
from jax.experimental.pallas import tpu_sc as plsc

N = 10000
E = 160000
D_IN = 256
D_HID = 128
D_OUT = 64

NDEG = 10240
NC = 2
NS = 16
CH = 128
EC = E // NC
CPC = EC // CH
NCHT = CPC // NS
RPTD = NDEG // NS
RPT = 624
RPT_LAST = 640
NB = 2

_mesh = plsc.VectorSubcoreMesh(core_axis_name="c", subcore_axis_name="s")


@functools.partial(
    pl.kernel,
    mesh=_mesh,
    out_type=jax.ShapeDtypeStruct((NC * NDEG,), jnp.float32),
    scratch_types=[
        pltpu.VMEM((NCHT + 1, 2, CH), jnp.int32),
        pltpu.VMEM((CH,), jnp.float32),
        pltpu.VMEM((RPTD,), jnp.float32),
        pltpu.VMEM_SHARED((NDEG,), jnp.float32),
        pltpu.SemaphoreType.DMA,
        pltpu.SemaphoreType.DMA,
    ],
)
def _deg_kernel(ep_hbm, out_hbm, idx_all, ones_v, zero_v, acc, d0, d1):
    c = lax.axis_index("c")
    s = lax.axis_index("s")
    tc0 = c * CPC + s * NCHT
    rbase = s * RPTD
    idx_cp = pltpu.async_copy(ep_hbm.at[pl.ds(tc0, NCHT)],
                              idx_all.at[pl.ds(0, NCHT)], d0)
    for i in range(RPTD // 16):
        zero_v[pl.ds(i * 16, 16)] = jnp.zeros((16,), jnp.float32)
    for i in range(CH // 16):
        ones_v[pl.ds(i * 16, 16)] = jnp.ones((16,), jnp.float32)
    pltpu.sync_copy(zero_v, acc.at[pl.ds(rbase, RPTD)])
    idx_cp.wait()

    @pl.when(s == 0)
    def _():
        pltpu.sync_copy(ep_hbm.at[pl.ds(c * CPC + NS * NCHT, 1)],
                        idx_all.at[pl.ds(NCHT, 1)])

    plsc.subcore_barrier()

    def sstart(ci, sem):
        pltpu.async_copy(ones_v, acc.at[idx_all.at[ci, 1]], sem, add=True)

    def swait(sem):
        pltpu.make_async_copy(ones_v, acc.at[idx_all.at[0, 1]], sem).wait()

    sstart(0, d0)
    sstart(1, d1)

    def body(j, carry):
        swait(d0)
        sstart(2 * j + 2, d0)
        swait(d1)
        sstart(2 * j + 3, d1)
        return carry

    lax.fori_loop(0, NCHT // 2 - 1, body, 0)
    swait(d0)
    sstart(NCHT - 1, d0)

    @pl.when(s == 0)
    def _():
        swait(d1)
        sstart(NCHT, d1)
        swait(d1)

    @pl.when(s != 0)
    def _():
        swait(d1)

    swait(d0)
    plsc.subcore_barrier()
    pltpu.sync_copy(acc.at[pl.ds(rbase, RPTD)],
                    out_hbm.at[pl.ds(c * NDEG + rbase, RPTD)])


@functools.partial(
    pl.kernel,
    mesh=_mesh,
    out_type=jax.ShapeDtypeStruct((NC * N, D_HID), jnp.float32),
    scratch_types=[
        pltpu.VMEM((NCHT + 1, 2, CH), jnp.int32),
        pltpu.VMEM((CH, D_HID), jnp.float32),
        pltpu.VMEM((CH, D_HID), jnp.float32),
        pltpu.VMEM_SHARED((N, D_HID), jnp.float32),
        pltpu.SemaphoreType.DMA,
        pltpu.SemaphoreType.DMA,
        pltpu.SemaphoreType.DMA,
        pltpu.SemaphoreType.DMA,
    ],
)
def _scatter_kernel(g_hbm, ep_hbm, out_hbm, idx_all,
                    rows0, rows1, acc, sg0, sg1, ss0, ss1):
    c = lax.axis_index("c")
    s = lax.axis_index("s")
    tc0 = c * CPC + s * NCHT
    rbase = s * RPT
    idx_cp = pltpu.async_copy(ep_hbm.at[pl.ds(tc0, NCHT)],
                              idx_all.at[pl.ds(0, NCHT)], sg0)
    @pl.when(s < NS - 1)
    def _():
        pltpu.sync_copy(g_hbm.at[pl.ds(rbase, RPT)],
                        acc.at[pl.ds(rbase, RPT)])

    @pl.when(s == NS - 1)
    def _():
        pltpu.sync_copy(g_hbm.at[pl.ds(rbase, RPT_LAST)],
                        acc.at[pl.ds(rbase, RPT_LAST)])

    idx_cp.wait()

    @pl.when(s == 0)
    def _():
        pltpu.sync_copy(ep_hbm.at[pl.ds(c * CPC + NS * NCHT, 1)],
                        idx_all.at[pl.ds(NCHT, 1)])

    plsc.subcore_barrier()

    rows = (rows0, rows1)
    sg = (sg0, sg1)
    ss = (ss0, ss1)

    def gstart(b, ci):
        pltpu.async_copy(g_hbm.at[idx_all.at[ci, 0]], rows[b], sg[b])

    def gwait(b):
        pltpu.make_async_copy(g_hbm.at[idx_all.at[0, 0]], rows[b],
                              sg[b]).wait()

    def sstart(b, ci):
        pltpu.async_copy(rows[b], acc.at[idx_all.at[ci, 1]], ss[b], add=True)

    def swait(b):
        pltpu.make_async_copy(rows[b], acc.at[idx_all.at[0, 1]],
                              ss[b]).wait()

    for b in range(NB):
        gstart(b, b)

    def body(j, carry):
        base = NB * j
        for b in range(NB):
            gwait(b)
            sstart(b, base + b)

        @pl.when(j < (NCHT - 1) // NB - 1)
        def _():
            for b in range(NB):
                swait(b)
                gstart(b, base + NB + b)

        return carry

    lax.fori_loop(0, (NCHT - 1) // NB, body, 0)
    swait(0)
    gstart(0, NCHT - 1)
    gwait(0)
    sstart(0, NCHT - 1)

    @pl.when(s == 0)
    def _():
        swait(1)
        gstart(1, NCHT)
        gwait(1)
        sstart(1, NCHT)
        swait(1)

    @pl.when(s != 0)
    def _():
        swait(1)

    swait(0)
    plsc.subcore_barrier()

    @pl.when(s < NS - 1)
    def _():
        pltpu.sync_copy(acc.at[pl.ds(rbase, RPT)],
                        out_hbm.at[pl.ds(c * N + rbase, RPT)])

    @pl.when(s == NS - 1)
    def _():
        pltpu.sync_copy(acc.at[pl.ds(rbase, RPT_LAST)],
                        out_hbm.at[pl.ds(c * N + rbase, RPT_LAST)])


BN = 1000


def _mm1_body(x_ref, w_ref, dinv_ref, g_ref):
    h = jnp.dot(x_ref[...], w_ref[...], preferred_element_type=jnp.float32)
    g_ref[...] = h * dinv_ref[...]


def _mm2_body(s_ref, g1_ref, dinv_ref, b1_ref, w_ref, g2_ref):
    agg = s_ref[0] + s_ref[1] - g1_ref[...]
    h = jnp.maximum(dinv_ref[...] * agg + b1_ref[...], 0.0)
    h2 = jnp.dot(h, w_ref[...], preferred_element_type=jnp.float32)
    g2_ref[...] = h2 * dinv_ref[...]


def _fin_body(t_ref, g2_ref, dinv_ref, bmu_ref, bls_ref, mu_ref, ls_ref):
    agg = t_ref[0] + t_ref[1] - g2_ref[...]
    o = dinv_ref[...] * agg
    mu_ref[...] = o[:, :D_OUT] + bmu_ref[...]
    ls_ref[...] = o[:, D_OUT:] + bls_ref[...]


def _col_spec():
    return pl.BlockSpec((BN, 1), lambda i: (i, 0))


def _row_spec(d):
    return pl.BlockSpec((BN, d), lambda i: (i, 0))


def _full_spec(r, d):
    return pl.BlockSpec((r, d), lambda i: (0, 0))


def _pair_spec(d):
    return pl.BlockSpec((NC, BN, d), lambda i: (0, i, 0))


def kernel(x, edge_index, W1, b1, Wmu, bmu, Wls, bls):
    ep = edge_index.reshape(2, E // CH, CH).swapaxes(0, 1)
    Wcat = jnp.concatenate([Wmu, Wls], axis=1)
    bmur = bmu.reshape(1, D_OUT)
    blsr = bls.reshape(1, D_OUT)
    b1r = b1.reshape(1, D_HID)

    degp = _deg_kernel(ep)
    deg = 1.0 + degp[:N] + degp[NDEG:NDEG + N]
    dinv = lax.rsqrt(deg).reshape(N, 1)

    grid = N // BN

    g1 = pl.pallas_call(
        _mm1_body,
        grid=(grid,),
        in_specs=[_row_spec(D_IN), _full_spec(D_IN, D_HID), _col_spec()],
        out_specs=_row_spec(D_HID),
        out_shape=jax.ShapeDtypeStruct((N, D_HID), jnp.float32),
    )(x, W1, dinv)

    s_pair = _scatter_kernel(g1, ep).reshape(NC, N, D_HID)

    g2 = pl.pallas_call(
        _mm2_body,
        grid=(grid,),
        in_specs=[_pair_spec(D_HID), _row_spec(D_HID), _col_spec(),
                  _full_spec(1, D_HID), _full_spec(D_HID, D_HID)],
        out_specs=_row_spec(D_HID),
        out_shape=jax.ShapeDtypeStruct((N, D_HID), jnp.float32),
    )(s_pair, g1, dinv, b1r, Wcat)

    t_pair = _scatter_kernel(g2, ep).reshape(NC, N, D_HID)

    mu, logstd = pl.pallas_call(
        _fin_body,
        grid=(grid,),
        in_specs=[_pair_spec(D_HID), _row_spec(D_HID), _col_spec(),
                  _full_spec(1, D_OUT), _full_spec(1, D_OUT)],
        out_specs=[_row_spec(D_OUT), _row_spec(D_OUT)],
        out_shape=[jax.ShapeDtypeStruct((N, D_OUT), jnp.float32),
                   jax.ShapeDtypeStruct((N, D_OUT), jnp.float32)],
    )(t_pair, g2, dinv, bmur, blsr)

    return (mu, logstd)
